# Initial kernel scaffold; baseline (speedup 1.0000x reference)
import jax
import jax.numpy as jnp
from jax.experimental import pallas as pl


def kernel(inputs, wte):
    raise NotImplementedError("write your pallas kernel here")


# speedup vs baseline: 1.4193x; 1.4193x over previous
import functools

import jax
import jax.numpy as jnp
from jax import lax
from jax.experimental import pallas as pl
from jax.experimental.pallas import tpu as pltpu
from jax.experimental.pallas import tpu_sc as plsc

D_MODEL = 2048
NUM_CORES = 2
NUM_SUBCORES = 16
NUM_WORKERS = NUM_CORES * NUM_SUBCORES
CHUNK = 16


@functools.partial(jax.jit, static_argnames=("total_rows",))
def _sc_gather(idx, wte, total_rows):
    rows_per_worker = total_rows // NUM_WORKERS
    n_chunks = rows_per_worker // CHUNK
    mesh = plsc.VectorSubcoreMesh(core_axis_name="c", subcore_axis_name="s")

    @functools.partial(
        pl.kernel,
        out_type=jax.ShapeDtypeStruct((total_rows, D_MODEL), jnp.float32),
        mesh=mesh,
        scratch_types=[
            pltpu.VMEM((n_chunks, CHUNK), jnp.int32),
            pltpu.VMEM((CHUNK, D_MODEL), jnp.float32),
            pltpu.SemaphoreType.DMA,
        ],
    )
    def gather_kernel(idx_hbm, wte_hbm, out_hbm, idx_v, rows_v, gsem):
        wid = lax.axis_index("s") * NUM_CORES + lax.axis_index("c")
        base = wid * rows_per_worker
        pltpu.sync_copy(idx_hbm.at[wid], idx_v)
        for j in range(n_chunks):
            pltpu.async_copy(wte_hbm.at[idx_v.at[j]], rows_v, gsem).wait()
            pltpu.sync_copy(rows_v, out_hbm.at[pl.ds(base + j * CHUNK, CHUNK)])

    return gather_kernel(idx.reshape(NUM_WORKERS, n_chunks, CHUNK), wte)


def kernel(inputs, wte):
    batch, seq = inputs.shape
    total = batch * seq
    idx = inputs.reshape(total).astype(jnp.int32)
    out = _sc_gather(idx, wte, total)
    return out.reshape(batch, seq, D_MODEL)


# speedup vs baseline: 1.5880x; 1.1188x over previous
import functools

import jax
import jax.numpy as jnp
from jax import lax
from jax.experimental import pallas as pl
from jax.experimental.pallas import tpu as pltpu
from jax.experimental.pallas import tpu_sc as plsc

D_MODEL = 2048
NUM_CORES = 2
NUM_SUBCORES = 16
NUM_WORKERS = NUM_CORES * NUM_SUBCORES
CHUNK = 16


@functools.partial(jax.jit, static_argnames=("total_rows",))
def _sc_gather(idx, wte, total_rows):
    rows_per_worker = total_rows // NUM_WORKERS
    n_chunks = rows_per_worker // CHUNK
    mesh = plsc.VectorSubcoreMesh(core_axis_name="c", subcore_axis_name="s")

    @functools.partial(
        pl.kernel,
        out_type=jax.ShapeDtypeStruct((total_rows, D_MODEL), jnp.float32),
        mesh=mesh,
        scratch_types=[
            pltpu.VMEM((n_chunks, CHUNK), jnp.int32),
            pltpu.VMEM((2, CHUNK, D_MODEL), jnp.float32),
            pltpu.SemaphoreType.DMA,
            pltpu.SemaphoreType.DMA,
            pltpu.SemaphoreType.DMA,
            pltpu.SemaphoreType.DMA,
        ],
    )
    def gather_kernel(idx_hbm, wte_hbm, out_hbm, idx_v, rows_v, g0, g1, s0, s1):
        wid = lax.axis_index("s") * NUM_CORES + lax.axis_index("c")
        base = wid * rows_per_worker
        gsem = (g0, g1)
        ssem = (s0, s1)
        pltpu.sync_copy(idx_hbm.at[wid], idx_v)

        def gather(j, slot):
            return pltpu.async_copy(
                wte_hbm.at[idx_v.at[j]], rows_v.at[slot], gsem[slot]
            )

        def store(j, slot):
            return pltpu.async_copy(
                rows_v.at[slot],
                out_hbm.at[pl.ds(base + j * CHUNK, CHUNK)],
                ssem[slot],
            )

        gd = {0: gather(0, 0)}
        sd = {}
        for j in range(n_chunks):
            slot = j % 2
            gd[j].wait()
            if j + 1 < n_chunks:
                if j >= 1:
                    sd[j - 1].wait()
                gd[j + 1] = gather(j + 1, 1 - slot)
            sd[j] = store(j, slot)
        if n_chunks >= 2:
            sd[n_chunks - 2].wait()
        sd[n_chunks - 1].wait()

    return gather_kernel(idx.reshape(NUM_WORKERS, n_chunks, CHUNK), wte)


def kernel(inputs, wte):
    batch, seq = inputs.shape
    total = batch * seq
    idx = inputs.reshape(total).astype(jnp.int32)
    out = _sc_gather(idx, wte, total)
    return out.reshape(batch, seq, D_MODEL)


# speedup vs baseline: 1.6743x; 1.0544x over previous
import functools

import jax
import jax.numpy as jnp
from jax import lax
from jax.experimental import pallas as pl
from jax.experimental.pallas import tpu as pltpu
from jax.experimental.pallas import tpu_sc as plsc

D_MODEL = 2048
NUM_CORES = 2
NUM_SUBCORES = 16
NUM_WORKERS = NUM_CORES * NUM_SUBCORES
CHUNK = 16
NBUF = 3


@functools.partial(jax.jit, static_argnames=("total_rows",))
def _sc_gather(idx, wte, total_rows):
    rows_per_worker = total_rows // NUM_WORKERS
    n_chunks = rows_per_worker // CHUNK
    mesh = plsc.VectorSubcoreMesh(core_axis_name="c", subcore_axis_name="s")

    @functools.partial(
        pl.kernel,
        out_type=jax.ShapeDtypeStruct((total_rows, D_MODEL), jnp.float32),
        mesh=mesh,
        scratch_types=[
            pltpu.VMEM((n_chunks, CHUNK), jnp.int32),
            pltpu.VMEM((NBUF, CHUNK, D_MODEL), jnp.float32),
            [pltpu.SemaphoreType.DMA] * NBUF,
            [pltpu.SemaphoreType.DMA] * NBUF,
        ],
    )
    def gather_kernel(idx_hbm, wte_hbm, out_hbm, idx_v, rows_v, gsem, ssem):
        wid = lax.axis_index("s") * NUM_CORES + lax.axis_index("c")
        base = wid * rows_per_worker
        pltpu.sync_copy(idx_hbm.at[wid], idx_v)

        def gather(j):
            slot = j % NBUF
            return pltpu.async_copy(
                wte_hbm.at[idx_v.at[j]], rows_v.at[slot], gsem[slot]
            )

        def store(j):
            slot = j % NBUF
            return pltpu.async_copy(
                rows_v.at[slot],
                out_hbm.at[pl.ds(base + j * CHUNK, CHUNK)],
                ssem[slot],
            )

        gd = {}
        sd = {}
        for j in range(min(NBUF - 1, n_chunks)):
            gd[j] = gather(j)
        for j in range(n_chunks):
            gd[j].wait()
            nxt = j + NBUF - 1
            if nxt < n_chunks:
                prev = nxt - NBUF
                if prev >= 0:
                    sd[prev].wait()
                gd[nxt] = gather(nxt)
            sd[j] = store(j)
        for j in range(max(0, n_chunks - NBUF), n_chunks):
            sd[j].wait()

    return gather_kernel(idx.reshape(NUM_WORKERS, n_chunks, CHUNK), wte)


def kernel(inputs, wte):
    batch, seq = inputs.shape
    total = batch * seq
    idx = inputs.reshape(total).astype(jnp.int32)
    out = _sc_gather(idx, wte, total)
    return out.reshape(batch, seq, D_MODEL)
